# baseline (speedup 1.0000x reference)
import jax
import jax.numpy as jnp
from jax.experimental import pallas as pl

N = 256
H = 256
A = 64
BI = 16


def _prep(t_ref, ts_ref, ss_ref, w1_ref, b1_ref, w2_ref, b2_ref,
          v_ref, c_ref, men_ref):
    tp2 = jax.lax.dot_general(
        t_ref[...], w2_ref[...], (((1,), (1,)), ((), ())),
        preferred_element_type=jnp.float32) + b2_ref[0, :][None, :]
    v_ref[...] = jax.lax.dot_general(
        tp2, w1_ref[...], (((1,), (0,)), ((), ())),
        preferred_element_type=jnp.float32)
    c_ref[...] = jnp.sum(tp2 * b1_ref[0, :][None, :], axis=1,
                         keepdims=True)
    m = jnp.logical_and(ts_ref[0, :] == 1.0,
                        ss_ref[0, :] == 0.0).astype(jnp.float32)
    men_ref[0, :] = m
    men_ref[1, :] = jnp.broadcast_to(jnp.sum(m), (N,))


def _attn_block(s_ref, v_ref, c_ref, men_ref, out_ref):
    i = pl.program_id(0)
    m = men_ref[0, :]
    en = men_ref[1, 0]
    s3 = s_ref[0]
    s2 = s3.reshape(BI * N, H)
    p = jax.lax.dot_general(
        s2, v_ref[...], (((1,), (1,)), ((), ())),
        preferred_element_type=jnp.float32)
    p3 = p.reshape(BI, N, BI)
    eye = (jax.lax.broadcasted_iota(jnp.int32, (BI, 1, BI), 0) ==
           jax.lax.broadcasted_iota(jnp.int32, (BI, 1, BI), 2)
           ).astype(jnp.float32)
    sm = jnp.sum(p3 * eye, axis=2)

    scale = en * jax.lax.rsqrt(jnp.float32(A))
    logits = (sm + c_ref[...]) * scale

    row_ids = i * BI + jax.lax.broadcasted_iota(jnp.int32, (BI, N), 0)
    col_ids = jax.lax.broadcasted_iota(jnp.int32, (BI, N), 1)
    off_diag = (row_ids != col_ids).astype(jnp.float32)
    m_rows = jnp.sum(jnp.where(row_ids == col_ids, m[None, :], 0.0), axis=1)
    num = jnp.exp(logits) * off_diag * m[None, :] * m_rows[:, None]
    den = jnp.sum(num, axis=1, keepdims=True)
    safe_den = jnp.where(den == 0.0, 1.0, den)
    score = num / safe_den

    out_ref[...] = jnp.sum(s3 * score[:, :, None], axis=1)


@jax.jit
def _edge_to_node_attention(spatial_ht_list, temporal_ht_list, ts_mask,
                            same_scene_mask, W1_w, W1_b, W2_w, W2_b):
    v, c, men = pl.pallas_call(
        _prep,
        out_shape=(
            jax.ShapeDtypeStruct((N, H), jnp.float32),
            jax.ShapeDtypeStruct((N, 1), jnp.float32),
            jax.ShapeDtypeStruct((2, N), jnp.float32),
        ),
    )(temporal_ht_list, ts_mask, same_scene_mask, W1_w, W1_b, W2_w, W2_b)

    return pl.pallas_call(
        _attn_block,
        grid=(N // BI,),
        in_specs=[
            pl.BlockSpec((1, BI, N, H), lambda i: (0, i, 0, 0)),
            pl.BlockSpec((BI, H), lambda i: (i, 0)),
            pl.BlockSpec((BI, 1), lambda i: (i, 0)),
            pl.BlockSpec((2, N), lambda i: (0, 0)),
        ],
        out_specs=pl.BlockSpec((BI, H), lambda i: (i, 0)),
        out_shape=jax.ShapeDtypeStruct((N, H), jnp.float32),
    )(spatial_ht_list, v, c, men)


def kernel(spatial_ht_list, temporal_ht_list, ts_mask, same_scene_mask,
           W1_w, W1_b, W2_w, W2_b):
    return _edge_to_node_attention(
        spatial_ht_list, temporal_ht_list,
        ts_mask.reshape(1, N), same_scene_mask.reshape(1, N),
        W1_w, W1_b.reshape(1, A), W2_w, W2_b.reshape(1, A))
